# baseline (speedup 1.0000x reference)
import functools

import jax
import jax.numpy as jnp
from jax import lax
from jax.experimental import pallas as pl
from jax.experimental.pallas import tpu as pltpu
from jax.experimental.pallas import tpu_sc as plsc

N_NODES = 100000
N_EDGES = 1600000
FDIM = 128
SDIM = 32
HDIM = 16
GDIM = 96
LDIM = 16
ROUNDS = 3

NC = 2
NS = 16
CHUNK = 512
BLK = 1

CHUNKS_PER_TILE = -(-N_EDGES // (CHUNK * 2 * BLK * NS)) * 2 * BLK
CHUNKS_TOTAL = CHUNKS_PER_TILE * NS
BLOCKS_PER_TILE = CHUNKS_PER_TILE // BLK
E_PAD = CHUNKS_TOTAL * CHUNK

ACC_ROWS = 100096
ZERO_PER_TILE = ACC_ROWS // NS
TRASH_ROW = N_NODES

ROW_T = 2000
GRID = N_NODES // ROW_T


def _dot(a, b):
    return jnp.dot(a, b, preferred_element_type=jnp.float32)


NGRP = 4
GRP = 500


def _gru_group(s_g, alo_g, ahi_g, gh_g, w, bih):
    gi = _dot(alo_g, w[:HDIM, :]) + _dot(ahi_g, w[HDIM:, :]) + bih
    r = jax.nn.sigmoid(gi[:, :SDIM] + gh_g[:, :SDIM])
    z = jax.nn.sigmoid(gi[:, SDIM:2 * SDIM] + gh_g[:, SDIM:2 * SDIM])
    n = jnp.tanh(gi[:, 2 * SDIM:] + r * gh_g[:, 2 * SDIM:])
    return s_g + (1.0 - z) * n + z * s_g


def _split_agg(a_ref, g):
    return jnp.concatenate([a_ref[:, SDIM * g:SDIM * g + HDIM],
                            a_ref[:, SDIM * g + HDIM:SDIM * g + SDIM]], axis=0)


def _emit_msg(m_list):
    los, his = [], []
    for m in m_list:
        los += [m[:GRP // 2, :HDIM], m[GRP // 2:, :HDIM]]
        his += [m[:GRP // 2, HDIM:], m[GRP // 2:, HDIM:]]
    return jnp.concatenate(los, axis=1), jnp.concatenate(his, axis=1)


def _tc_init_body(x_ref, inW_ref, inb_ref, mW_ref, mb_ref, whhT_ref, bhh_ref,
                  state_ref, mlo_ref, mhi_ref, gh_ref):
    sts, ms, ghs = [], [], []
    for g in range(NGRP):
        xs = x_ref[GRP * g:GRP * (g + 1), :]
        st = jnp.maximum(_dot(xs, inW_ref[...]) + inb_ref[...], 0.0)
        sts.append(st)
        ms.append(jnp.maximum(_dot(st, mW_ref[...]) + mb_ref[...], 0.0))
        ghs.append(_dot(st, whhT_ref[...]) + bhh_ref[...])
    state_ref[0] = jnp.concatenate(sts, axis=1)
    gh_ref[0] = jnp.concatenate(ghs, axis=1)
    mlo_ref[0], mhi_ref[0] = _emit_msg(ms)


def _tc_mid_body(state_ref, alo_ref, ahi_ref, gh_ref, wihT_ref, bih_ref,
                 mW_ref, mb_ref, whhT_ref, bhh_ref,
                 nstate_ref, mlo_ref, mhi_ref, ghn_ref):
    w = wihT_ref[...]
    news, ms, ghs = [], [], []
    for g in range(NGRP):
        new = _gru_group(state_ref[0, :, SDIM * g:SDIM * (g + 1)],
                         _split_agg(alo_ref[0], g), _split_agg(ahi_ref[0], g),
                         gh_ref[0, :, GDIM * g:GDIM * (g + 1)], w, bih_ref[...])
        news.append(new)
        ms.append(jnp.maximum(_dot(new, mW_ref[...]) + mb_ref[...], 0.0))
        ghs.append(_dot(new, whhT_ref[...]) + bhh_ref[...])
    nstate_ref[0] = jnp.concatenate(news, axis=1)
    ghn_ref[0] = jnp.concatenate(ghs, axis=1)
    mlo_ref[0], mhi_ref[0] = _emit_msg(ms)


def _tc_final_body(state_ref, alo_ref, ahi_ref, gh_ref, wihT_ref, bih_ref,
                   muW_ref, mub_ref, lsW_ref, lsb_ref, mu_ref, ls_ref):
    w = wihT_ref[...]
    mus, lss = [], []
    for g in range(NGRP):
        new = _gru_group(state_ref[0, :, SDIM * g:SDIM * (g + 1)],
                         _split_agg(alo_ref[0], g), _split_agg(ahi_ref[0], g),
                         gh_ref[0, :, GDIM * g:GDIM * (g + 1)], w, bih_ref[...])
        mus.append(_dot(new, muW_ref[...]) + mub_ref[...])
        lss.append(_dot(new, lsW_ref[...]) + lsb_ref[...])
    mu_ref[...] = jnp.concatenate(mus, axis=0)
    ls_ref[...] = jnp.concatenate(lss, axis=0)


STATE_P = (GRID, GRP, NGRP * SDIM)
GH_P = (GRID, GRP, NGRP * GDIM)
MSG_P = (GRID, GRP // 2, 8 * HDIM)


def _blk3(shape3):
    _, h, w = shape3
    return pl.BlockSpec((1, h, w), lambda i: (i, 0, 0))


def _blk(shape_div):
    h, w = shape_div
    return pl.BlockSpec((h, w), lambda i: (i, 0))


def _full_spec(shape):
    return pl.BlockSpec(shape, lambda i: (0,) * len(shape))


def _sds(*shape):
    return jax.ShapeDtypeStruct(shape, jnp.float32)


_STATE_B = _blk3(STATE_P)
_GH_B = _blk3(GH_P)
_MSG_B = _blk3(MSG_P)
_PACK_OUTS = [_sds(*STATE_P), _sds(*MSG_P), _sds(*MSG_P), _sds(*GH_P)]
_PACK_OUT_SPECS = [_STATE_B, _MSG_B, _MSG_B, _GH_B]


def _tc_init(x, inW, inb, mW, mb, whhT, bhh):
    return pl.pallas_call(
        _tc_init_body,
        grid=(GRID,),
        in_specs=[_blk((ROW_T, FDIM)), _full_spec((FDIM, SDIM)),
                  _full_spec((1, SDIM)), _full_spec((SDIM, SDIM)),
                  _full_spec((1, SDIM)), _full_spec((SDIM, GDIM)),
                  _full_spec((1, GDIM))],
        out_specs=_PACK_OUT_SPECS,
        out_shape=_PACK_OUTS,
    )(x, inW, inb, mW, mb, whhT, bhh)


def _tc_mid(state, alo, ahi, gh, wihT, bih, mW, mb, whhT, bhh):
    return pl.pallas_call(
        _tc_mid_body,
        grid=(GRID,),
        in_specs=[_STATE_B, _MSG_B, _MSG_B, _GH_B,
                  _full_spec((SDIM, GDIM)), _full_spec((1, GDIM)),
                  _full_spec((SDIM, SDIM)), _full_spec((1, SDIM)),
                  _full_spec((SDIM, GDIM)), _full_spec((1, GDIM))],
        out_specs=_PACK_OUT_SPECS,
        out_shape=_PACK_OUTS,
    )(state, alo, ahi, gh, wihT, bih, mW, mb, whhT, bhh)


def _tc_final(state, alo, ahi, gh, wihT, bih, muW, mub, lsW, lsb):
    return pl.pallas_call(
        _tc_final_body,
        grid=(GRID,),
        in_specs=[_STATE_B, _MSG_B, _MSG_B, _GH_B,
                  _full_spec((SDIM, GDIM)), _full_spec((1, GDIM)),
                  _full_spec((SDIM, LDIM)), _full_spec((1, LDIM)),
                  _full_spec((SDIM, LDIM)), _full_spec((1, LDIM))],
        out_specs=[_blk((ROW_T, LDIM)), _blk((ROW_T, LDIM))],
        out_shape=[_sds(N_NODES, LDIM), _sds(N_NODES, LDIM)],
    )(state, alo, ahi, gh, wihT, bih, muW, mub, lsW, lsb)


def _sc_body(mlo_hbm, mhi_hbm, idx_hbm, zero_hbm, alo_hbm, ahi_hbm,
             acc, iv0, iv1, rows0, rows1, gsem, ssem):
    c = lax.axis_index("c")
    s = lax.axis_index("s")
    stripe = s * ZERO_PER_TILE

    pltpu.sync_copy(zero_hbm.at[pl.ds(stripe, ZERO_PER_TILE)],
                    acc.at[pl.ds(stripe, ZERO_PER_TILE)])
    plsc.subcore_barrier()

    def _accumulate(msg_ref):
        cbase = s * CHUNKS_PER_TILE

        def _fire_block(iv, rows, blk0):
            pltpu.sync_copy(idx_hbm.at[pl.ds(blk0, BLK)], iv)
            for j in range(BLK):
                pltpu.async_copy(msg_ref.at[iv.at[j, 0]], rows.at[j], gsem)

        def _drain_g_fire_s(iv, rows):
            for j in range(BLK):
                pltpu.make_async_copy(msg_ref.at[iv.at[j, 0]], rows.at[j],
                                      gsem).wait()
                pltpu.async_copy(rows.at[j], acc.at[iv.at[j, 1]], ssem,
                                 add=True)

        def _drain_s(iv, rows):
            for j in range(BLK):
                pltpu.make_async_copy(rows.at[j], acc.at[iv.at[j, 1]],
                                      ssem).wait()

        _fire_block(iv0, rows0, cbase)
        _fire_block(iv1, rows1, cbase + BLK)

        def _pair(p, _):
            b0 = cbase + (2 * p) * BLK
            _drain_g_fire_s(iv0, rows0)
            _drain_g_fire_s(iv1, rows1)
            _drain_s(iv0, rows0)
            _fire_block(iv0, rows0, b0 + 2 * BLK)
            _drain_s(iv1, rows1)
            _fire_block(iv1, rows1, b0 + 3 * BLK)
            return 0
        lax.fori_loop(0, BLOCKS_PER_TILE // 2 - 1, _pair, 0)
        _drain_g_fire_s(iv0, rows0)
        _drain_g_fire_s(iv1, rows1)
        _drain_s(iv0, rows0)
        _drain_s(iv1, rows1)

    @pl.when(c == 0)
    def _():
        _accumulate(mlo_hbm)

    @pl.when(c == 1)
    def _():
        _accumulate(mhi_hbm)

    plsc.subcore_barrier()

    @pl.when(c == 0)
    def _():
        pltpu.sync_copy(acc.at[pl.ds(stripe, ZERO_PER_TILE)],
                        alo_hbm.at[pl.ds(stripe, ZERO_PER_TILE)])

    @pl.when(c == 1)
    def _():
        pltpu.sync_copy(acc.at[pl.ds(stripe, ZERO_PER_TILE)],
                        ahi_hbm.at[pl.ds(stripe, ZERO_PER_TILE)])


@functools.cache
def _sc_aggregate_fn():
    return pl.kernel(
        _sc_body,
        out_type=[_sds(ACC_ROWS, HDIM), _sds(ACC_ROWS, HDIM)],
        mesh=plsc.VectorSubcoreMesh(core_axis_name="c", subcore_axis_name="s"),
        scratch_types=[
            pltpu.VMEM_SHARED((ACC_ROWS, HDIM), jnp.float32),
            pltpu.VMEM((BLK, 2, CHUNK), jnp.int32),
            pltpu.VMEM((BLK, 2, CHUNK), jnp.int32),
            pltpu.VMEM((BLK, CHUNK, HDIM), jnp.float32),
            pltpu.VMEM((BLK, CHUNK, HDIM), jnp.float32),
            pltpu.SemaphoreType.DMA,
            pltpu.SemaphoreType.DMA,
        ],
        compiler_params=pltpu.CompilerParams(use_tc_tiling_on_sc=False),
    )


def _sc_aggregate(mlo_p, mhi_p, idx_comb, zeros):
    alo, ahi = _sc_aggregate_fn()(mlo_p.reshape(N_NODES, HDIM),
                                  mhi_p.reshape(N_NODES, HDIM),
                                  idx_comb, zeros)
    return (alo[:N_NODES].reshape(MSG_P), ahi[:N_NODES].reshape(MSG_P))


def kernel(x, edge_index, input_W, input_b, msg_W, msg_b, gru_wih, gru_whh,
           gru_bih, gru_bhh, mu_W, mu_b, ls_W, ls_b):
    pad = E_PAD - N_EDGES

    def _perm(n):
        return (250 * (n // ROW_T) + n % 250) * 8 + (n % ROW_T) // 250

    src = jnp.concatenate([_perm(edge_index[0]), jnp.zeros((pad,), jnp.int32)])
    dst = jnp.concatenate([_perm(edge_index[1]),
                           jnp.full((pad,), TRASH_ROW, jnp.int32)])
    idx_comb = jnp.stack([src.reshape(CHUNKS_TOTAL, CHUNK),
                          dst.reshape(CHUNKS_TOTAL, CHUNK)], axis=1)
    zeros = jnp.zeros((ACC_ROWS, HDIM), jnp.float32)

    inb = input_b.reshape(1, SDIM)
    mb = msg_b.reshape(ROUNDS, 1, SDIM)
    bih = gru_bih.reshape(ROUNDS, 1, GDIM)
    bhh = gru_bhh.reshape(ROUNDS, 1, GDIM)
    wihT = jnp.transpose(gru_wih, (0, 2, 1))
    whhT = jnp.transpose(gru_whh, (0, 2, 1))
    mub = mu_b.reshape(1, LDIM)
    lsb = ls_b.reshape(1, LDIM)

    state, mlo, mhi, gh = _tc_init(x, input_W, inb, msg_W[0], mb[0],
                                   whhT[0], bhh[0])
    for r in range(ROUNDS):
        alo, ahi = _sc_aggregate(mlo, mhi, idx_comb, zeros)
        if r < ROUNDS - 1:
            state, mlo, mhi, gh = _tc_mid(state, alo, ahi, gh, wihT[r], bih[r],
                                          msg_W[r + 1], mb[r + 1],
                                          whhT[r + 1], bhh[r + 1])
        else:
            mu, ls = _tc_final(state, alo, ahi, gh, wihT[r], bih[r],
                               mu_W, mub, ls_W, lsb)
    return (mu, ls)
